# Initial kernel scaffold; baseline (speedup 1.0000x reference)
import jax
import jax.numpy as jnp
from jax.experimental import pallas as pl


def kernel(x, edge_index, Wq, Wk, Wv, Wo, ln_scale, ln_bias):
    raise NotImplementedError("write your pallas kernel here")


# speedup vs baseline: 16.4779x; 16.4779x over previous
import functools

import jax
import jax.numpy as jnp
import numpy as np
from jax import lax
from jax.experimental import pallas as pl
from jax.experimental.pallas import tpu as pltpu
from jax.experimental.pallas import tpu_sc as plsc

N = 10000
E = 320000
D = 128
H = 8
DH = D // H
MDW = D + 16

NC = 2
NS = 16
NW = NC * NS
EPW = E // NW
EB = 40
NCHUNK = EPW // EB
RPW = N // NS
RCH = 25


def _edge_body(q_hbm, k_hbm, v_hbm, src_hbm, dst_hbm, md_hbm,
               src_v, dst_v, qrows, krows, vrows, msg, ebuf, sem, md_sh):
    c = lax.axis_index("c")
    s = lax.axis_index("s")
    wid = c * NS + s

    zero16 = jnp.zeros((DH,), jnp.float32)

    def zrow(r, carry):
        for j in range(MDW // DH):
            ebuf[r, pl.ds(j * DH, DH)] = zero16
        return carry

    lax.fori_loop(0, RCH, zrow, 0)
    row0 = s * RPW
    for t in range(RPW // RCH):
        pltpu.sync_copy(ebuf, md_sh.at[pl.ds(row0 + t * RCH, RCH)])
    plsc.subcore_barrier()

    lane = lax.iota(jnp.int32, DH)
    ebase = wid * EPW

    def chunk(i, carry):
        base = ebase + i * EB
        pltpu.sync_copy(src_hbm.at[pl.ds(base, EB)], src_v)
        pltpu.sync_copy(dst_hbm.at[pl.ds(base, EB)], dst_v)
        pltpu.async_copy(q_hbm.at[dst_v], qrows, sem).wait()
        pltpu.async_copy(k_hbm.at[src_v], krows, sem).wait()
        pltpu.async_copy(v_hbm.at[src_v], vrows, sem).wait()

        def edge(e, carry2):
            exvec = jnp.zeros((DH,), jnp.float32)
            for h in range(H):
                qh = qrows[e, pl.ds(h * DH, DH)]
                kh = krows[e, pl.ds(h * DH, DH)]
                sc = jnp.sum(qh * kh)
                exv = jnp.exp(jnp.broadcast_to(sc, (DH,)) * 0.25)
                vh = vrows[e, pl.ds(h * DH, DH)]
                msg[e, pl.ds(h * DH, DH)] = exv * vh
                exvec = jnp.where(lane == h, exv, exvec)
            msg[e, pl.ds(D, DH)] = exvec
            return carry2

        lax.fori_loop(0, EB, edge, 0)
        pltpu.sync_copy(msg, md_sh.at[dst_v], add=True)
        return carry

    lax.fori_loop(0, NCHUNK, chunk, 0)
    plsc.subcore_barrier()

    for t in range(RPW // RCH):
        r0 = row0 + t * RCH
        pltpu.sync_copy(md_sh.at[pl.ds(r0, RCH)], ebuf)
        pltpu.sync_copy(ebuf, md_hbm.at[c].at[pl.ds(r0, RCH)])


_edge_attn = pl.kernel(
    _edge_body,
    out_type=jax.ShapeDtypeStruct((NC, N, MDW), jnp.float32),
    mesh=plsc.VectorSubcoreMesh(core_axis_name="c", subcore_axis_name="s",
                                num_cores=NC, num_subcores=NS),
    compiler_params=pltpu.CompilerParams(use_tc_tiling_on_sc=False,
                                         needs_layout_passes=False),
    scratch_types=[
        pltpu.VMEM((EB,), jnp.int32),
        pltpu.VMEM((EB,), jnp.int32),
        pltpu.VMEM((EB, D), jnp.float32),
        pltpu.VMEM((EB, D), jnp.float32),
        pltpu.VMEM((EB, D), jnp.float32),
        pltpu.VMEM((EB, MDW), jnp.float32),
        pltpu.VMEM((RCH, MDW), jnp.float32),
        pltpu.SemaphoreType.DMA,
        pltpu.VMEM_SHARED((N, MDW), jnp.float32),
    ],
)


BM = 400


def _qkv_body(x_ref, wq_ref, wk_ref, wv_ref, q_ref, k_ref, v_ref):
    xb = x_ref[...]
    q_ref[...] = jnp.dot(xb, wq_ref[...], preferred_element_type=jnp.float32)
    k_ref[...] = jnp.dot(xb, wk_ref[...], preferred_element_type=jnp.float32)
    v_ref[...] = jnp.dot(xb, wv_ref[...], preferred_element_type=jnp.float32)


_qkv = pl.pallas_call(
    _qkv_body,
    grid=(N // BM,),
    in_specs=[
        pl.BlockSpec((BM, D), lambda i: (i, 0)),
        pl.BlockSpec((D, D), lambda i: (0, 0)),
        pl.BlockSpec((D, D), lambda i: (0, 0)),
        pl.BlockSpec((D, D), lambda i: (0, 0)),
    ],
    out_specs=[pl.BlockSpec((BM, D), lambda i: (i, 0))] * 3,
    out_shape=[jax.ShapeDtypeStruct((N, D), jnp.float32)] * 3,
)

def _final_body(x_ref, md_ref, wo_ref, g_ref, b_ref, o_ref):
    m = md_ref[0] + md_ref[1]
    rep = (lax.broadcasted_iota(jnp.int32, (H, D), 1) // DH
           == lax.broadcasted_iota(jnp.int32, (H, D), 0)).astype(jnp.float32)
    den = jnp.dot(m[:, D:D + H], rep,
                  preferred_element_type=jnp.float32)
    agg = m[:, :D] / (den + 1e-9)
    out = jnp.dot(agg, wo_ref[...], preferred_element_type=jnp.float32)
    hh = x_ref[...] + out
    mu = jnp.mean(hh, axis=-1, keepdims=True)
    var = jnp.mean((hh - mu) ** 2, axis=-1, keepdims=True)
    o_ref[...] = (hh - mu) / jnp.sqrt(var + 1e-5) * g_ref[...] + b_ref[...]


_final = pl.pallas_call(
    _final_body,
    grid=(N // BM,),
    in_specs=[
        pl.BlockSpec((BM, D), lambda i: (i, 0)),
        pl.BlockSpec((NC, BM, MDW), lambda i: (0, i, 0)),
        pl.BlockSpec((D, D), lambda i: (0, 0)),
        pl.BlockSpec((1, D), lambda i: (0, 0)),
        pl.BlockSpec((1, D), lambda i: (0, 0)),
    ],
    out_specs=pl.BlockSpec((BM, D), lambda i: (i, 0)),
    out_shape=jax.ShapeDtypeStruct((N, D), jnp.float32),
)


def kernel(x, edge_index, Wq, Wk, Wv, Wo, ln_scale, ln_bias):
    src = edge_index[0]
    dst = edge_index[1]
    q, k, v = _qkv(x, Wq, Wk, Wv)
    md = _edge_attn(q, k, v, src, dst)
    return _final(x, md, Wo, ln_scale.reshape(1, D), ln_bias.reshape(1, D))


# speedup vs baseline: 18.5698x; 1.1270x over previous
import functools

import jax
import jax.numpy as jnp
import numpy as np
from jax import lax
from jax.experimental import pallas as pl
from jax.experimental.pallas import tpu as pltpu
from jax.experimental.pallas import tpu_sc as plsc

N = 10000
E = 320000
D = 128
H = 8
DH = D // H
MDW = D + 16

NC = 2
NS = 16
NW = NC * NS
EPW = E // NW
EB = 40
NCHUNK = EPW // EB
RPW = N // NS
RCH = 25


def _edge_body(q_hbm, k_hbm, v_hbm, src_hbm, dst_hbm, md_hbm,
               src_v, dst_v, qrows, krows, vrows, msg, ebuf, sem, md_sh):
    c = lax.axis_index("c")
    s = lax.axis_index("s")
    wid = c * NS + s

    zero16 = jnp.zeros((DH,), jnp.float32)

    def zrow(r, carry):
        for j in range(MDW // DH):
            ebuf[r, pl.ds(j * DH, DH)] = zero16
        return carry

    lax.fori_loop(0, RCH, zrow, 0)
    row0 = s * RPW
    for t in range(RPW // RCH):
        pltpu.sync_copy(ebuf, md_sh.at[pl.ds(row0 + t * RCH, RCH)])
    plsc.subcore_barrier()

    lane = lax.iota(jnp.int32, DH)
    ebase = wid * EPW

    def chunk(i, carry):
        base = ebase + i * EB
        pltpu.sync_copy(src_hbm.at[pl.ds(base, EB)], src_v)
        pltpu.sync_copy(dst_hbm.at[pl.ds(base, EB)], dst_v)
        cq = pltpu.async_copy(q_hbm.at[dst_v], qrows, sem)
        ck = pltpu.async_copy(k_hbm.at[src_v], krows, sem)
        cv = pltpu.async_copy(v_hbm.at[src_v], vrows, sem)
        cq.wait()
        ck.wait()
        cv.wait()

        def edge(e, carry2):
            exvec = jnp.zeros((DH,), jnp.float32)
            for h in range(H):
                qh = qrows[e, pl.ds(h * DH, DH)]
                kh = krows[e, pl.ds(h * DH, DH)]
                sc = jnp.sum(qh * kh)
                exv = jnp.exp(jnp.broadcast_to(sc, (DH,)) * 0.25)
                vh = vrows[e, pl.ds(h * DH, DH)]
                msg[e, pl.ds(h * DH, DH)] = exv * vh
                exvec = jnp.where(lane == h, exv, exvec)
            msg[e, pl.ds(D, DH)] = exvec
            return carry2

        lax.fori_loop(0, EB, edge, 0)
        pltpu.sync_copy(msg, md_sh.at[dst_v], add=True)
        return carry

    lax.fori_loop(0, NCHUNK, chunk, 0)
    plsc.subcore_barrier()

    for t in range(RPW // RCH):
        r0 = row0 + t * RCH
        pltpu.sync_copy(md_sh.at[pl.ds(r0, RCH)], ebuf)
        pltpu.sync_copy(ebuf, md_hbm.at[c].at[pl.ds(r0, RCH)])


_edge_attn = pl.kernel(
    _edge_body,
    out_type=jax.ShapeDtypeStruct((NC, N, MDW), jnp.float32),
    mesh=plsc.VectorSubcoreMesh(core_axis_name="c", subcore_axis_name="s",
                                num_cores=NC, num_subcores=NS),
    compiler_params=pltpu.CompilerParams(use_tc_tiling_on_sc=False,
                                         needs_layout_passes=False),
    scratch_types=[
        pltpu.VMEM((EB,), jnp.int32),
        pltpu.VMEM((EB,), jnp.int32),
        pltpu.VMEM((EB, D), jnp.float32),
        pltpu.VMEM((EB, D), jnp.float32),
        pltpu.VMEM((EB, D), jnp.float32),
        pltpu.VMEM((EB, MDW), jnp.float32),
        pltpu.VMEM((RCH, MDW), jnp.float32),
        pltpu.SemaphoreType.DMA,
        pltpu.VMEM_SHARED((N, MDW), jnp.float32),
    ],
)


BM = 400


def _qkv_body(x_ref, wq_ref, wk_ref, wv_ref, q_ref, k_ref, v_ref):
    xb = x_ref[...]
    q_ref[...] = jnp.dot(xb, wq_ref[...], preferred_element_type=jnp.float32)
    k_ref[...] = jnp.dot(xb, wk_ref[...], preferred_element_type=jnp.float32)
    v_ref[...] = jnp.dot(xb, wv_ref[...], preferred_element_type=jnp.float32)


_qkv = pl.pallas_call(
    _qkv_body,
    grid=(N // BM,),
    in_specs=[
        pl.BlockSpec((BM, D), lambda i: (i, 0)),
        pl.BlockSpec((D, D), lambda i: (0, 0)),
        pl.BlockSpec((D, D), lambda i: (0, 0)),
        pl.BlockSpec((D, D), lambda i: (0, 0)),
    ],
    out_specs=[pl.BlockSpec((BM, D), lambda i: (i, 0))] * 3,
    out_shape=[jax.ShapeDtypeStruct((N, D), jnp.float32)] * 3,
)

def _final_body(x_ref, md_ref, wo_ref, g_ref, b_ref, o_ref):
    m = md_ref[0] + md_ref[1]
    rep = (lax.broadcasted_iota(jnp.int32, (H, D), 1) // DH
           == lax.broadcasted_iota(jnp.int32, (H, D), 0)).astype(jnp.float32)
    den = jnp.dot(m[:, D:D + H], rep,
                  preferred_element_type=jnp.float32)
    agg = m[:, :D] / (den + 1e-9)
    out = jnp.dot(agg, wo_ref[...], preferred_element_type=jnp.float32)
    hh = x_ref[...] + out
    mu = jnp.mean(hh, axis=-1, keepdims=True)
    var = jnp.mean((hh - mu) ** 2, axis=-1, keepdims=True)
    o_ref[...] = (hh - mu) / jnp.sqrt(var + 1e-5) * g_ref[...] + b_ref[...]


_final = pl.pallas_call(
    _final_body,
    grid=(N // BM,),
    in_specs=[
        pl.BlockSpec((BM, D), lambda i: (i, 0)),
        pl.BlockSpec((NC, BM, MDW), lambda i: (0, i, 0)),
        pl.BlockSpec((D, D), lambda i: (0, 0)),
        pl.BlockSpec((1, D), lambda i: (0, 0)),
        pl.BlockSpec((1, D), lambda i: (0, 0)),
    ],
    out_specs=pl.BlockSpec((BM, D), lambda i: (i, 0)),
    out_shape=jax.ShapeDtypeStruct((N, D), jnp.float32),
)


def kernel(x, edge_index, Wq, Wk, Wv, Wo, ln_scale, ln_bias):
    src = edge_index[0]
    dst = edge_index[1]
    q, k, v = _qkv(x, Wq, Wk, Wv)
    md = _edge_attn(q, k, v, src, dst)
    return _final(x, md, Wo, ln_scale.reshape(1, D), ln_bias.reshape(1, D))
